# baseline (speedup 1.0000x reference)
import functools

import jax
import jax.numpy as jnp
from jax.experimental import pallas as pl
from jax.experimental.pallas import tpu as pltpu

B, T, D_WP, HID = 16384, 20, 3, 512
BB = 256


def _body(wp_ref, wt_ref, pb_ref, emb_ref, out_ref):
    wp = wp_ref[...]
    comb = emb_ref[...] + pb_ref[...]
    for t in range(T):
        acc = comb[t : t + 1, :]
        for d in range(D_WP):
            acc = acc + wp[:, 3 * t + d : 3 * t + d + 1] * wt_ref[d : d + 1, :]
        out_ref[:, t, :] = acc


@functools.partial(jax.jit)
def kernel(waypoints, proj_w, proj_b, emb_table):
    wp2d = waypoints.reshape(B, T * D_WP)
    wt = proj_w.T
    pb = proj_b.reshape(1, HID)
    grid = (B // BB,)
    out = pl.pallas_call(
        _body,
        grid=grid,
        in_specs=[
            pl.BlockSpec((BB, T * D_WP), lambda i: (i, 0)),
            pl.BlockSpec((D_WP, HID), lambda i: (0, 0)),
            pl.BlockSpec((1, HID), lambda i: (0, 0)),
            pl.BlockSpec((T, HID), lambda i: (0, 0)),
        ],
        out_specs=pl.BlockSpec((BB, T, HID), lambda i: (i, 0, 0)),
        out_shape=jax.ShapeDtypeStruct((B, T, HID), jnp.float32),
        compiler_params=pltpu.CompilerParams(
            dimension_semantics=("arbitrary",),
        ),
    )(wp2d, wt, pb, emb_table)
    return out
